# Initial kernel scaffold; baseline (speedup 1.0000x reference)
import jax
import jax.numpy as jnp
from jax.experimental import pallas as pl


def kernel(x, W_enc, b_enc, W_dec, b_dec):
    raise NotImplementedError("write your pallas kernel here")


# speedup vs baseline: 1.0808x; 1.0808x over previous
import functools

import jax
import jax.numpy as jnp
from jax import lax
from jax.experimental import pallas as pl
from jax.experimental.pallas import tpu as pltpu
from jax.experimental.pallas import tpu_sc as plsc

D_IN = 768
N_LAT = 49152
N_TOK = 2048
TOPK = 64


_LAT_BLK = 512


def _enc_body(x_ref, w_ref, b_ref, out_ref):
    acts = lax.dot_general(
        x_ref[...], w_ref[...],
        dimension_numbers=(((1,), (1,)), ((), ())),
        preferred_element_type=jnp.float32,
    )
    out_ref[...] = jnp.maximum(acts + b_ref[...], 0.0)


def _encode(sae_in, W_enc, b_enc):
    grid = (N_LAT // _LAT_BLK,)
    return pl.pallas_call(
        _enc_body,
        grid=grid,
        in_specs=[
            pl.BlockSpec((N_TOK, D_IN), lambda j: (0, 0)),
            pl.BlockSpec((_LAT_BLK, D_IN), lambda j: (j, 0)),
            pl.BlockSpec((1, _LAT_BLK), lambda j: (0, j)),
        ],
        out_specs=pl.BlockSpec((N_TOK, _LAT_BLK), lambda j: (0, j)),
        out_shape=jax.ShapeDtypeStruct((N_TOK, N_LAT), jnp.float32),
    )(sae_in, W_enc, b_enc.reshape(1, N_LAT))


_NC = 2
_NS = 16
_NW = _NC * _NS
_TPW = N_TOK // _NW
_CCH = D_IN // 16


def _dec_body(wdec_hbm, idx_hbm, acts_hbm, bdec_hbm, out_hbm,
              idx_v, acts_v, rows_v, bdec_v, outrow_v, sem):
    wid = lax.axis_index("s") * _NC + lax.axis_index("c")
    base = wid * _TPW
    pltpu.sync_copy(bdec_hbm, bdec_v)

    def tok_body(t, carry):
        tok = base + t
        pltpu.sync_copy(idx_hbm.at[tok], idx_v)
        pltpu.sync_copy(acts_hbm.at[tok], acts_v)
        pltpu.async_copy(wdec_hbm.at[idx_v], rows_v, sem).wait()
        for c in range(_CCH):
            outrow_v[pl.ds(c * 16, 16)] = bdec_v[pl.ds(c * 16, 16)]

        def k_body(k, carry2):
            a_b = acts_v[pl.ds(k * 16, 16)]
            for c in range(_CCH):
                plsc.addupdate(outrow_v.at[pl.ds(c * 16, 16)],
                               a_b * rows_v[k, pl.ds(c * 16, 16)])
            return carry2

        lax.fori_loop(0, TOPK, k_body, 0, unroll=False)
        pltpu.sync_copy(outrow_v, out_hbm.at[tok])
        return carry

    lax.fori_loop(0, _TPW, tok_body, 0, unroll=False)


def _decode(W_dec, top_idx, top_acts, b_dec):
    acts_exp = jnp.broadcast_to(
        top_acts[:, :, None], (N_TOK, TOPK, 16)).reshape(N_TOK, TOPK * 16)
    mesh = plsc.VectorSubcoreMesh(core_axis_name="c", subcore_axis_name="s")
    f = functools.partial(
        pl.kernel,
        mesh=mesh,
        out_type=jax.ShapeDtypeStruct((N_TOK, D_IN), jnp.float32),
        scratch_types=[
            pltpu.VMEM((TOPK,), jnp.int32),
            pltpu.VMEM((TOPK * 16,), jnp.float32),
            pltpu.VMEM((TOPK, D_IN), jnp.float32),
            pltpu.VMEM((D_IN,), jnp.float32),
            pltpu.VMEM((D_IN,), jnp.float32),
            pltpu.SemaphoreType.DMA,
        ],
    )(_dec_body)
    return f(W_dec, top_idx, acts_exp, b_dec)


def kernel(x, W_enc, b_enc, W_dec, b_dec):
    sae_in = x - b_dec[None, :]
    pre_acts = _encode(sae_in, W_enc, b_enc)
    top_acts, top_idx = lax.top_k(pre_acts, TOPK)
    sae_out = _decode(W_dec, top_idx, top_acts, b_dec)
    e = x - sae_out
    total_variance = jnp.sum((x - jnp.mean(x, axis=0)) ** 2)
    fvu = jnp.sum(e * e) / total_variance
    return sae_out, top_acts, top_idx, fvu


# speedup vs baseline: 8.1625x; 7.5521x over previous
import functools

import jax
import jax.numpy as jnp
from jax import lax
from jax.experimental import pallas as pl
from jax.experimental.pallas import tpu as pltpu
from jax.experimental.pallas import tpu_sc as plsc

D_IN = 768
N_LAT = 49152
N_TOK = 2048
TOPK = 64
GRP = 128
NGRP = N_LAT // GRP
CHK = 16
NCHK = GRP // CHK


_LAT_BLK = 512
_GPB = _LAT_BLK // GRP
_CPB = _LAT_BLK // CHK


def _enc_body(x_ref, w_ref, b_ref, out_ref, gm128_ref):
    acts = lax.dot_general(
        x_ref[...], w_ref[...],
        dimension_numbers=(((1,), (1,)), ((), ())),
        preferred_element_type=jnp.float32,
    )
    acts = jnp.maximum(acts + b_ref[...], 0.0)
    out_ref[...] = acts
    gm128_ref[...] = jnp.max(
        acts.reshape(N_TOK, _GPB, GRP), axis=2).reshape(1, N_TOK, _GPB)


def _encode(sae_in, W_enc, b_enc):
    grid = (N_LAT // _LAT_BLK,)
    return pl.pallas_call(
        _enc_body,
        grid=grid,
        in_specs=[
            pl.BlockSpec((N_TOK, D_IN), lambda j: (0, 0)),
            pl.BlockSpec((_LAT_BLK, D_IN), lambda j: (j, 0)),
            pl.BlockSpec((1, _LAT_BLK), lambda j: (0, j)),
        ],
        out_specs=[
            pl.BlockSpec((N_TOK, _LAT_BLK), lambda j: (0, j)),
            pl.BlockSpec((1, N_TOK, _GPB), lambda j: (j, 0, 0)),
        ],
        out_shape=[
            jax.ShapeDtypeStruct((N_TOK, N_LAT), jnp.float32),
            jax.ShapeDtypeStruct((N_LAT // _LAT_BLK, N_TOK, _GPB),
                                 jnp.float32),
        ],
    )(sae_in, W_enc, b_enc.reshape(1, N_LAT))


_TB = 256


def _gsel_body(gm_ref, vals_ref, gids_ref):
    gm = gm_ref[...]
    lane = lax.broadcasted_iota(jnp.int32, (_TB, NGRP), 1)
    vals = []
    gids = []
    for _ in range(TOPK):
        m = jnp.max(gm, axis=1, keepdims=True)
        hit = gm == m
        idx = jnp.min(jnp.where(hit, lane, NGRP), axis=1, keepdims=True)
        vals.append(m)
        gids.append(idx)
        gm = jnp.where(lane == idx, -1.0, gm)
    vals_ref[...] = jnp.concatenate(vals, axis=1)
    gids_ref[...] = jnp.concatenate(gids, axis=1)


def _group_select(gm128):
    grid = (N_TOK // _TB,)
    return pl.pallas_call(
        _gsel_body,
        grid=grid,
        in_specs=[pl.BlockSpec((_TB, NGRP), lambda i: (i, 0))],
        out_specs=[
            pl.BlockSpec((_TB, TOPK), lambda i: (i, 0)),
            pl.BlockSpec((_TB, TOPK), lambda i: (i, 0)),
        ],
        out_shape=[
            jax.ShapeDtypeStruct((N_TOK, TOPK), jnp.float32),
            jax.ShapeDtypeStruct((N_TOK, TOPK), jnp.int32),
        ],
    )(gm128)


_NC = 2
_NS = 16
_NW = _NC * _NS
_TPW = N_TOK // _NW
_CCH = D_IN // 16


def _sc_body(pa_hbm, vals_hbm, gids_hbm, wdec_hbm, bdec_hbm,
             topv_hbm, topi_hbm, out_hbm,
             gid_v, cur_v, rowidx_v, vals_v, topv_v, topi_v,
             wrows_v, bdec_v, outrow_v, sem):
    wid = lax.axis_index("s") * _NC + lax.axis_index("c")
    base = wid * _TPW
    pltpu.sync_copy(bdec_hbm, bdec_v)
    lanes16 = lax.iota(jnp.int32, 16)
    big = jnp.int32(255)

    def _extract(ref, pos):
        seg = ref[pl.ds((pos >> 4) * 16, 16)]
        return jnp.max(jnp.where(lanes16 == (pos & 15), seg,
                                 jnp.full((16,), -2147483647,
                                          seg.dtype)))

    def _insert(ref, pos, val):
        off = (pos >> 4) * 16
        seg = ref[pl.ds(off, 16)]
        ref[pl.ds(off, 16)] = jnp.where(lanes16 == (pos & 15),
                                        jnp.full((16,), val, seg.dtype), seg)

    def tok_body(t, carry):
        tok = base + t
        pltpu.sync_copy(gids_hbm.at[tok], gid_v)
        pltpu.sync_copy(vals_hbm.at[tok], cur_v)
        for j in range(4):
            rowidx_v[pl.ds(j * 16, 16)] = (
                gid_v[pl.ds(j * 16, 16)] + tok * NGRP)
        pltpu.async_copy(pa_hbm.at[rowidx_v], vals_v, sem).wait()

        def pop_body(i, carry2):
            c0 = cur_v[pl.ds(0, 16)]
            c1 = cur_v[pl.ds(16, 16)]
            c2 = cur_v[pl.ds(32, 16)]
            c3 = cur_v[pl.ds(48, 16)]
            v = jnp.max(jnp.maximum(jnp.maximum(c0, c1),
                                    jnp.maximum(c2, c3)))
            slot = big
            for j, cj in enumerate((c3, c2, c1, c0)):
                pos = jnp.min(jnp.where(cj == v, lanes16, big))
                slot = jnp.where(pos < 16, pos + (3 - j) * 16, slot)
            cbest = big
            lane = big
            for c in range(NCHK):
                ch = vals_v[slot, pl.ds(c * CHK, CHK)]
                pos = jnp.min(jnp.where(ch == v, lanes16, big))
                hit = jnp.logical_and(pos < 16, cbest >= 16)
                cbest = jnp.where(hit, jnp.int32(c), cbest)
                lane = jnp.where(hit, pos, lane)
            gsel = _extract(gid_v, slot)
            gidx = gsel * GRP + cbest * CHK + lane
            _insert(topv_v, i, v)
            _insert(topi_v, i, gidx)
            ch = vals_v[slot, pl.ds(cbest * CHK, CHK)]
            vals_v[slot, pl.ds(cbest * CHK, CHK)] = jnp.where(
                lanes16 == lane, jnp.float32(-1.0), ch)
            ncur = jnp.full((16,), -1.0, jnp.float32)
            for c in range(NCHK):
                ncur = jnp.maximum(ncur, vals_v[slot, pl.ds(c * CHK, CHK)])
            _insert(cur_v, slot, jnp.max(ncur))
            return carry2

        lax.fori_loop(0, TOPK, pop_body, 0, unroll=False)
        pltpu.sync_copy(topv_v, topv_hbm.at[tok])
        pltpu.sync_copy(topi_v, topi_hbm.at[tok])
        pltpu.async_copy(wdec_hbm.at[topi_v], wrows_v, sem).wait()
        for c in range(_CCH):
            outrow_v[pl.ds(c * 16, 16)] = bdec_v[pl.ds(c * 16, 16)]

        def k_body(k, carry2):
            a_b = jnp.full((16,), _extract(topv_v, k), jnp.float32)
            for c in range(_CCH):
                plsc.addupdate(outrow_v.at[pl.ds(c * 16, 16)],
                               a_b * wrows_v[k, pl.ds(c * 16, 16)])
            return carry2

        lax.fori_loop(0, TOPK, k_body, 0, unroll=False)
        pltpu.sync_copy(outrow_v, out_hbm.at[tok])
        return carry

    lax.fori_loop(0, _TPW, tok_body, 0, unroll=False)


def _sc_select_decode(pre_acts, vals, gids, W_dec, b_dec):
    pa_rows = pre_acts.reshape(N_TOK * NGRP, GRP)
    mesh = plsc.VectorSubcoreMesh(core_axis_name="c", subcore_axis_name="s")
    f = functools.partial(
        pl.kernel,
        mesh=mesh,
        compiler_params=pltpu.CompilerParams(needs_layout_passes=False),
        out_type=[
            jax.ShapeDtypeStruct((N_TOK, TOPK), jnp.float32),
            jax.ShapeDtypeStruct((N_TOK, TOPK), jnp.int32),
            jax.ShapeDtypeStruct((N_TOK, D_IN), jnp.float32),
        ],
        scratch_types=[
            pltpu.VMEM((TOPK,), jnp.int32),
            pltpu.VMEM((TOPK,), jnp.float32),
            pltpu.VMEM((TOPK,), jnp.int32),
            pltpu.VMEM((TOPK, GRP), jnp.float32),
            pltpu.VMEM((TOPK,), jnp.float32),
            pltpu.VMEM((TOPK,), jnp.int32),
            pltpu.VMEM((TOPK, D_IN), jnp.float32),
            pltpu.VMEM((D_IN,), jnp.float32),
            pltpu.VMEM((D_IN,), jnp.float32),
            pltpu.SemaphoreType.DMA,
        ],
    )(_sc_body)
    return f(pa_rows, vals, gids, W_dec, b_dec)


def kernel(x, W_enc, b_enc, W_dec, b_dec):
    sae_in = x - b_dec[None, :]
    pre_acts, gm128_3d = _encode(sae_in, W_enc, b_enc)
    gm128 = gm128_3d.transpose(1, 0, 2).reshape(N_TOK, NGRP)
    vals, gids = _group_select(gm128)
    top_acts, top_idx, sae_out = _sc_select_decode(
        pre_acts, vals, gids, W_dec, b_dec)
    e = x - sae_out
    total_variance = jnp.sum((x - jnp.mean(x, axis=0)) ** 2)
    fvu = jnp.sum(e * e) / total_variance
    return sae_out, top_acts, top_idx, fvu


# speedup vs baseline: 9.1391x; 1.1196x over previous
import functools

import jax
import jax.numpy as jnp
from jax import lax
from jax.experimental import pallas as pl
from jax.experimental.pallas import tpu as pltpu
from jax.experimental.pallas import tpu_sc as plsc

D_IN = 768
N_LAT = 49152
N_TOK = 2048
TOPK = 64
GRP = 128
NGRP = N_LAT // GRP
CHK = 16
NCHK = GRP // CHK


_LAT_BLK = 512
_GPB = _LAT_BLK // GRP
_CPB = _LAT_BLK // CHK


def _enc_body(x_ref, w_ref, b_ref, out_ref, gm128_ref):
    acts = lax.dot_general(
        x_ref[...], w_ref[...],
        dimension_numbers=(((1,), (1,)), ((), ())),
        preferred_element_type=jnp.float32,
    )
    acts = jnp.maximum(acts + b_ref[...], 0.0)
    out_ref[...] = acts
    gm128_ref[...] = jnp.max(
        acts.reshape(N_TOK, _GPB, GRP), axis=2).reshape(1, N_TOK, _GPB)


def _encode(sae_in, W_enc, b_enc):
    grid = (N_LAT // _LAT_BLK,)
    return pl.pallas_call(
        _enc_body,
        grid=grid,
        in_specs=[
            pl.BlockSpec((N_TOK, D_IN), lambda j: (0, 0)),
            pl.BlockSpec((_LAT_BLK, D_IN), lambda j: (j, 0)),
            pl.BlockSpec((1, _LAT_BLK), lambda j: (0, j)),
        ],
        out_specs=[
            pl.BlockSpec((N_TOK, _LAT_BLK), lambda j: (0, j)),
            pl.BlockSpec((1, N_TOK, _GPB), lambda j: (j, 0, 0)),
        ],
        out_shape=[
            jax.ShapeDtypeStruct((N_TOK, N_LAT), jnp.float32),
            jax.ShapeDtypeStruct((N_LAT // _LAT_BLK, N_TOK, _GPB),
                                 jnp.float32),
        ],
    )(sae_in, W_enc, b_enc.reshape(1, N_LAT))


_TB = 256


def _gsel_body(gm_ref, vals_ref, gids_ref):
    gm = gm_ref[...]
    lane = lax.broadcasted_iota(jnp.int32, (_TB, NGRP), 1)
    vals = []
    gids = []
    for _ in range(TOPK):
        m = jnp.max(gm, axis=1, keepdims=True)
        hit = gm == m
        idx = jnp.min(jnp.where(hit, lane, NGRP), axis=1, keepdims=True)
        vals.append(m)
        gids.append(idx)
        gm = jnp.where(lane == idx, -1.0, gm)
    vals_ref[...] = jnp.concatenate(vals, axis=1)
    gids_ref[...] = jnp.concatenate(gids, axis=1)


def _group_select(gm128):
    grid = (N_TOK // _TB,)
    return pl.pallas_call(
        _gsel_body,
        grid=grid,
        in_specs=[pl.BlockSpec((_TB, NGRP), lambda i: (i, 0))],
        out_specs=[
            pl.BlockSpec((_TB, TOPK), lambda i: (i, 0)),
            pl.BlockSpec((_TB, TOPK), lambda i: (i, 0)),
        ],
        out_shape=[
            jax.ShapeDtypeStruct((N_TOK, TOPK), jnp.float32),
            jax.ShapeDtypeStruct((N_TOK, TOPK), jnp.int32),
        ],
    )(gm128)


_NC = 2
_NS = 16
_NW = _NC * _NS
_TPW = N_TOK // _NW
_CCH = D_IN // 16


def _sc_body(pa_hbm, vals_hbm, gids_hbm, wdec_hbm, bdec_hbm,
             topv_hbm, topi_hbm, out_hbm,
             gidslab_v, gid_v, cur0_v, cur1_v, rowidx_v,
             vals0_v, vals1_v, topv0_v, topv1_v, topi0_v, topi1_v,
             wrows0_v, wrows1_v, bdec_v, outrow_v,
             pa_sem0, pa_sem1, wd_sem0, wd_sem1, cu_sem0, cu_sem1):
    wid = lax.axis_index("s") * _NC + lax.axis_index("c")
    base = wid * _TPW
    pltpu.sync_copy(bdec_hbm, bdec_v)
    pltpu.sync_copy(gids_hbm.at[pl.ds(base, _TPW)], gidslab_v)
    lanes16 = lax.iota(jnp.int32, 16)
    big = jnp.int32(255)
    vals_b = (vals0_v, vals1_v)
    cur_b = (cur0_v, cur1_v)
    topv_b = (topv0_v, topv1_v)
    topi_b = (topi0_v, topi1_v)
    wrows_b = (wrows0_v, wrows1_v)
    pa_sems = (pa_sem0, pa_sem1)
    wd_sems = (wd_sem0, wd_sem1)
    cu_sems = (cu_sem0, cu_sem1)

    def _extract(ref, pos):
        seg = ref[pl.ds((pos >> 4) * 16, 16)]
        return jnp.max(jnp.where(lanes16 == (pos & 15), seg,
                                 jnp.full((16,), -2147483647,
                                          seg.dtype)))

    def _insert(ref, pos, val):
        off = (pos >> 4) * 16
        seg = ref[pl.ds(off, 16)]
        ref[pl.ds(off, 16)] = jnp.where(lanes16 == (pos & 15),
                                        jnp.full((16,), val, seg.dtype), seg)

    def issue_pa(tl, buf):
        tok = base + tl
        for j in range(4):
            rowidx_v[pl.ds(j * 16, 16)] = (
                gidslab_v[tl, pl.ds(j * 16, 16)] + tok * NGRP)
        pltpu.async_copy(vals_hbm.at[tok], cur_b[buf], cu_sems[buf])
        return pltpu.async_copy(pa_hbm.at[rowidx_v], vals_b[buf],
                                pa_sems[buf])

    def pops(tl, buf):
        vals_v = vals_b[buf]
        cur_v = cur_b[buf]
        topv_v = topv_b[buf]
        topi_v = topi_b[buf]
        for j in range(4):
            gid_v[pl.ds(j * 16, 16)] = gidslab_v[tl, pl.ds(j * 16, 16)]

        def pop_body(i, carry2):
            c0 = cur_v[pl.ds(0, 16)]
            c1 = cur_v[pl.ds(16, 16)]
            c2 = cur_v[pl.ds(32, 16)]
            c3 = cur_v[pl.ds(48, 16)]
            v = jnp.max(jnp.maximum(jnp.maximum(c0, c1),
                                    jnp.maximum(c2, c3)))
            slot = big
            for j, cj in enumerate((c3, c2, c1, c0)):
                pos = jnp.min(jnp.where(cj == v, lanes16, big))
                slot = jnp.where(pos < 16, pos + (3 - j) * 16, slot)
            cbest = big
            lane = big
            for c in range(NCHK):
                ch = vals_v[slot, pl.ds(c * CHK, CHK)]
                pos = jnp.min(jnp.where(ch == v, lanes16, big))
                hit = jnp.logical_and(pos < 16, cbest >= 16)
                cbest = jnp.where(hit, jnp.int32(c), cbest)
                lane = jnp.where(hit, pos, lane)
            gsel = _extract(gid_v, slot)
            gidx = gsel * GRP + cbest * CHK + lane
            _insert(topv_v, i, v)
            _insert(topi_v, i, gidx)
            ch = vals_v[slot, pl.ds(cbest * CHK, CHK)]
            vals_v[slot, pl.ds(cbest * CHK, CHK)] = jnp.where(
                lanes16 == lane, jnp.float32(-1.0), ch)
            ncur = jnp.full((16,), -1.0, jnp.float32)
            for c in range(NCHK):
                ncur = jnp.maximum(ncur, vals_v[slot, pl.ds(c * CHK, CHK)])
            _insert(cur_v, slot, jnp.max(ncur))
            return carry2

        lax.fori_loop(0, TOPK, pop_body, 0, unroll=False)
        tok = base + tl
        pltpu.sync_copy(topv_v, topv_hbm.at[tok])
        pltpu.sync_copy(topi_v, topi_hbm.at[tok])

    def accumulate(tl, buf):
        topv_v = topv_b[buf]
        wrows_v = wrows_b[buf]
        for c in range(_CCH):
            outrow_v[pl.ds(c * 16, 16)] = bdec_v[pl.ds(c * 16, 16)]

        def k_body(k, carry2):
            a_b = jnp.full((16,), _extract(topv_v, k), jnp.float32)
            for c in range(_CCH):
                plsc.addupdate(outrow_v.at[pl.ds(c * 16, 16)],
                               a_b * wrows_v[k, pl.ds(c * 16, 16)])
            return carry2

        lax.fori_loop(0, TOPK, k_body, 0, unroll=False)
        pltpu.sync_copy(outrow_v, out_hbm.at[base + tl])

    def wait_pa(buf):
        pltpu.make_async_copy(vals_hbm.at[base], cur_b[buf],
                              cu_sems[buf]).wait()
        pltpu.make_async_copy(pa_hbm.at[rowidx_v], vals_b[buf],
                              pa_sems[buf]).wait()

    def wait_wd(buf):
        pltpu.make_async_copy(wdec_hbm.at[topi_b[buf]], wrows_b[buf],
                              wd_sems[buf]).wait()

    def issue_wd(buf):
        pltpu.async_copy(wdec_hbm.at[topi_b[buf]], wrows_b[buf],
                         wd_sems[buf])

    issue_pa(0, 0)
    wait_pa(0)
    pops(0, 0)
    issue_wd(0)
    issue_pa(1, 1)

    def step(t, buf):
        wait_pa(buf)
        pops(t, buf)
        issue_wd(buf)
        issue_pa(jnp.minimum(t + 1, _TPW - 1), 1 - buf)
        wait_wd(1 - buf)
        accumulate(t - 1, 1 - buf)

    def loop_body(i, carry):
        step(1 + 2 * i, 1)
        step(2 + 2 * i, 0)
        return carry

    lax.fori_loop(0, (_TPW - 2) // 2, loop_body, 0, unroll=False)
    wait_pa(1)
    pops(_TPW - 1, 1)
    issue_wd(1)
    wait_wd(0)
    accumulate(_TPW - 2, 0)
    wait_wd(1)
    accumulate(_TPW - 1, 1)


def _sc_select_decode(pre_acts, vals, gids, W_dec, b_dec):
    pa_rows = pre_acts.reshape(N_TOK * NGRP, GRP)
    mesh = plsc.VectorSubcoreMesh(core_axis_name="c", subcore_axis_name="s")
    f = functools.partial(
        pl.kernel,
        mesh=mesh,
        compiler_params=pltpu.CompilerParams(needs_layout_passes=False),
        out_type=[
            jax.ShapeDtypeStruct((N_TOK, TOPK), jnp.float32),
            jax.ShapeDtypeStruct((N_TOK, TOPK), jnp.int32),
            jax.ShapeDtypeStruct((N_TOK, D_IN), jnp.float32),
        ],
        scratch_types=[
            pltpu.VMEM((_TPW, TOPK), jnp.int32),
            pltpu.VMEM((TOPK,), jnp.int32),
            pltpu.VMEM((TOPK,), jnp.float32),
            pltpu.VMEM((TOPK,), jnp.float32),
            pltpu.VMEM((TOPK,), jnp.int32),
            pltpu.VMEM((TOPK, GRP), jnp.float32),
            pltpu.VMEM((TOPK, GRP), jnp.float32),
            pltpu.VMEM((TOPK,), jnp.float32),
            pltpu.VMEM((TOPK,), jnp.float32),
            pltpu.VMEM((TOPK,), jnp.int32),
            pltpu.VMEM((TOPK,), jnp.int32),
            pltpu.VMEM((TOPK, D_IN), jnp.float32),
            pltpu.VMEM((TOPK, D_IN), jnp.float32),
            pltpu.VMEM((D_IN,), jnp.float32),
            pltpu.VMEM((D_IN,), jnp.float32),
            pltpu.SemaphoreType.DMA,
            pltpu.SemaphoreType.DMA,
            pltpu.SemaphoreType.DMA,
            pltpu.SemaphoreType.DMA,
            pltpu.SemaphoreType.DMA,
            pltpu.SemaphoreType.DMA,
        ],
    )(_sc_body)
    return f(pa_rows, vals, gids, W_dec, b_dec)


def kernel(x, W_enc, b_enc, W_dec, b_dec):
    sae_in = x - b_dec[None, :]
    pre_acts, gm128_3d = _encode(sae_in, W_enc, b_enc)
    gm128 = gm128_3d.transpose(1, 0, 2).reshape(N_TOK, NGRP)
    vals, gids = _group_select(gm128)
    top_acts, top_idx, sae_out = _sc_select_decode(
        pre_acts, vals, gids, W_dec, b_dec)
    e = x - sae_out
    total_variance = jnp.sum((x - jnp.mean(x, axis=0)) ** 2)
    fvu = jnp.sum(e * e) / total_variance
    return sae_out, top_acts, top_idx, fvu


# speedup vs baseline: 9.3190x; 1.0197x over previous
import functools

import jax
import jax.numpy as jnp
from jax import lax
from jax.experimental import pallas as pl
from jax.experimental.pallas import tpu as pltpu
from jax.experimental.pallas import tpu_sc as plsc

D_IN = 768
N_LAT = 49152
N_TOK = 2048
TOPK = 64
GRP = 128
NGRP = N_LAT // GRP
CHK = 16
NCHK = GRP // CHK


_LAT_BLK = 512
_GPB = _LAT_BLK // GRP
_CPB = _LAT_BLK // CHK


def _enc_body(x_ref, w_ref, b_ref, out_ref, gm128_ref):
    acts = lax.dot_general(
        x_ref[...], w_ref[...],
        dimension_numbers=(((1,), (1,)), ((), ())),
        preferred_element_type=jnp.float32,
    )
    acts = jnp.maximum(acts + b_ref[...], 0.0)
    out_ref[...] = acts
    gm128_ref[...] = jnp.max(
        acts.reshape(N_TOK, _GPB, GRP), axis=2).reshape(1, N_TOK, _GPB)


def _encode(sae_in, W_enc, b_enc):
    grid = (N_LAT // _LAT_BLK,)
    return pl.pallas_call(
        _enc_body,
        grid=grid,
        in_specs=[
            pl.BlockSpec((N_TOK, D_IN), lambda j: (0, 0)),
            pl.BlockSpec((_LAT_BLK, D_IN), lambda j: (j, 0)),
            pl.BlockSpec((1, _LAT_BLK), lambda j: (0, j)),
        ],
        out_specs=[
            pl.BlockSpec((N_TOK, _LAT_BLK), lambda j: (0, j)),
            pl.BlockSpec((1, N_TOK, _GPB), lambda j: (j, 0, 0)),
        ],
        out_shape=[
            jax.ShapeDtypeStruct((N_TOK, N_LAT), jnp.float32),
            jax.ShapeDtypeStruct((N_LAT // _LAT_BLK, N_TOK, _GPB),
                                 jnp.float32),
        ],
    )(sae_in, W_enc, b_enc.reshape(1, N_LAT))


_TB = 256


def _gsel_body(gm_ref, vals_ref, gids_ref):
    gm = gm_ref[...]
    lane = lax.broadcasted_iota(jnp.int32, (_TB, NGRP), 1)
    vals = []
    gids = []
    for _ in range(TOPK):
        m = jnp.max(gm, axis=1, keepdims=True)
        hit = gm == m
        idx = jnp.min(jnp.where(hit, lane, NGRP), axis=1, keepdims=True)
        vals.append(m)
        gids.append(idx)
        gm = jnp.where(lane == idx, -1.0, gm)
    vals_ref[...] = jnp.concatenate(vals, axis=1)
    gids_ref[...] = jnp.concatenate(gids, axis=1)


def _group_select(gm128):
    grid = (N_TOK // _TB,)
    return pl.pallas_call(
        _gsel_body,
        grid=grid,
        in_specs=[pl.BlockSpec((_TB, NGRP), lambda i: (i, 0))],
        out_specs=[
            pl.BlockSpec((_TB, TOPK), lambda i: (i, 0)),
            pl.BlockSpec((_TB, TOPK), lambda i: (i, 0)),
        ],
        out_shape=[
            jax.ShapeDtypeStruct((N_TOK, TOPK), jnp.float32),
            jax.ShapeDtypeStruct((N_TOK, TOPK), jnp.int32),
        ],
    )(gm128)


_NC = 2
_NS = 16
_NW = _NC * _NS
_TPW = N_TOK // _NW
_CCH = D_IN // 16


def _sc_body(pa_hbm, vals_hbm, gids_hbm, wdec_hbm, bdec_hbm,
             topv_hbm, topi_hbm, out_hbm,
             gidslab_v, gid_v, cur0_v, cur1_v, rowidx_v,
             vals0_v, vals1_v, topv0_v, topv1_v, topi0_v, topi1_v,
             wrows0_v, wrows1_v, bdec_v, outrow_v,
             pa_sem0, pa_sem1, wd_sem0, wd_sem1, cu_sem0, cu_sem1):
    wid = lax.axis_index("s") * _NC + lax.axis_index("c")
    base = wid * _TPW
    pltpu.sync_copy(bdec_hbm, bdec_v)
    pltpu.sync_copy(gids_hbm.at[pl.ds(base, _TPW)], gidslab_v)
    lanes16 = lax.iota(jnp.int32, 16)
    big = jnp.int32(255)
    vals_b = (vals0_v, vals1_v)
    cur_b = (cur0_v, cur1_v)
    topv_b = (topv0_v, topv1_v)
    topi_b = (topi0_v, topi1_v)
    wrows_b = (wrows0_v, wrows1_v)
    pa_sems = (pa_sem0, pa_sem1)
    wd_sems = (wd_sem0, wd_sem1)
    cu_sems = (cu_sem0, cu_sem1)

    def _extract(ref, pos):
        seg = ref[pl.ds((pos >> 4) * 16, 16)]
        return jnp.max(jnp.where(lanes16 == (pos & 15), seg,
                                 jnp.full((16,), -2147483647,
                                          seg.dtype)))

    def _insert(ref, pos, val):
        off = (pos >> 4) * 16
        seg = ref[pl.ds(off, 16)]
        ref[pl.ds(off, 16)] = jnp.where(lanes16 == (pos & 15),
                                        jnp.full((16,), val, seg.dtype), seg)

    def issue_pa(tl, buf):
        tok = base + tl
        for j in range(4):
            rowidx_v[pl.ds(j * 16, 16)] = (
                gidslab_v[tl, pl.ds(j * 16, 16)] + tok * NGRP)
        pltpu.async_copy(vals_hbm.at[tok], cur_b[buf], cu_sems[buf])
        return pltpu.async_copy(pa_hbm.at[rowidx_v], vals_b[buf],
                                pa_sems[buf])

    def pops(tl, buf):
        vals_v = vals_b[buf]
        cur_v = cur_b[buf]
        topv_v = topv_b[buf]
        topi_v = topi_b[buf]
        for j in range(4):
            gid_v[pl.ds(j * 16, 16)] = gidslab_v[tl, pl.ds(j * 16, 16)]

        bigi = jnp.int32(1 << 30)

        def pop_body(i, carry2):
            c0 = cur_v[pl.ds(0, 16)]
            c1 = cur_v[pl.ds(16, 16)]
            c2 = cur_v[pl.ds(32, 16)]
            c3 = cur_v[pl.ds(48, 16)]
            v = jnp.max(jnp.maximum(jnp.maximum(c0, c1),
                                    jnp.maximum(c2, c3)))
            sc = jnp.full((16,), bigi, jnp.int32)
            for j, cj in enumerate((c0, c1, c2, c3)):
                sc = jnp.minimum(sc, jnp.where(cj == v,
                                               j * 16 + lanes16, bigi))
            slot = jnp.minimum(jnp.min(sc), jnp.int32(TOPK - 1))
            lc = jnp.full((16,), bigi, jnp.int32)
            for c in range(NCHK):
                ch = vals_v[slot, pl.ds(c * CHK, CHK)]
                lc = jnp.minimum(lc, jnp.where(ch == v,
                                               c * 256 + lanes16, bigi))
            code = jnp.min(lc)
            cbest = jnp.minimum(code >> 8, jnp.int32(NCHK - 1))
            lane = code & 15
            gsel = _extract(gid_v, slot)
            gidx = gsel * GRP + cbest * CHK + lane
            _insert(topv_v, i, v)
            _insert(topi_v, i, gidx)
            ch = vals_v[slot, pl.ds(cbest * CHK, CHK)]
            vals_v[slot, pl.ds(cbest * CHK, CHK)] = jnp.where(
                lanes16 == lane, jnp.float32(-1.0), ch)
            ncur = jnp.full((16,), -1.0, jnp.float32)
            for c in range(NCHK):
                ncur = jnp.maximum(ncur, vals_v[slot, pl.ds(c * CHK, CHK)])
            _insert(cur_v, slot, jnp.max(ncur))
            return carry2

        lax.fori_loop(0, TOPK, pop_body, 0, unroll=False)
        tok = base + tl
        pltpu.sync_copy(topv_v, topv_hbm.at[tok])
        pltpu.sync_copy(topi_v, topi_hbm.at[tok])

    def accumulate(tl, buf):
        topv_v = topv_b[buf]
        wrows_v = wrows_b[buf]
        for c in range(_CCH):
            outrow_v[pl.ds(c * 16, 16)] = bdec_v[pl.ds(c * 16, 16)]

        def k_body(k, carry2):
            a_b = jnp.full((16,), _extract(topv_v, k), jnp.float32)
            for c in range(_CCH):
                plsc.addupdate(outrow_v.at[pl.ds(c * 16, 16)],
                               a_b * wrows_v[k, pl.ds(c * 16, 16)])
            return carry2

        lax.fori_loop(0, TOPK, k_body, 0, unroll=False)
        pltpu.sync_copy(outrow_v, out_hbm.at[base + tl])

    def wait_pa(buf):
        pltpu.make_async_copy(vals_hbm.at[base], cur_b[buf],
                              cu_sems[buf]).wait()
        pltpu.make_async_copy(pa_hbm.at[rowidx_v], vals_b[buf],
                              pa_sems[buf]).wait()

    def wait_wd(buf):
        pltpu.make_async_copy(wdec_hbm.at[topi_b[buf]], wrows_b[buf],
                              wd_sems[buf]).wait()

    def issue_wd(buf):
        pltpu.async_copy(wdec_hbm.at[topi_b[buf]], wrows_b[buf],
                         wd_sems[buf])

    issue_pa(0, 0)
    wait_pa(0)
    pops(0, 0)
    issue_wd(0)
    issue_pa(1, 1)

    def step(t, buf):
        wait_pa(buf)
        pops(t, buf)
        issue_wd(buf)
        issue_pa(jnp.minimum(t + 1, _TPW - 1), 1 - buf)
        wait_wd(1 - buf)
        accumulate(t - 1, 1 - buf)

    def loop_body(i, carry):
        step(1 + 2 * i, 1)
        step(2 + 2 * i, 0)
        return carry

    lax.fori_loop(0, (_TPW - 2) // 2, loop_body, 0, unroll=False)
    wait_pa(1)
    pops(_TPW - 1, 1)
    issue_wd(1)
    wait_wd(0)
    accumulate(_TPW - 2, 0)
    wait_wd(1)
    accumulate(_TPW - 1, 1)


def _sc_select_decode(pre_acts, vals, gids, W_dec, b_dec):
    pa_rows = pre_acts.reshape(N_TOK * NGRP, GRP)
    mesh = plsc.VectorSubcoreMesh(core_axis_name="c", subcore_axis_name="s")
    f = functools.partial(
        pl.kernel,
        mesh=mesh,
        compiler_params=pltpu.CompilerParams(needs_layout_passes=False),
        out_type=[
            jax.ShapeDtypeStruct((N_TOK, TOPK), jnp.float32),
            jax.ShapeDtypeStruct((N_TOK, TOPK), jnp.int32),
            jax.ShapeDtypeStruct((N_TOK, D_IN), jnp.float32),
        ],
        scratch_types=[
            pltpu.VMEM((_TPW, TOPK), jnp.int32),
            pltpu.VMEM((TOPK,), jnp.int32),
            pltpu.VMEM((TOPK,), jnp.float32),
            pltpu.VMEM((TOPK,), jnp.float32),
            pltpu.VMEM((TOPK,), jnp.int32),
            pltpu.VMEM((TOPK, GRP), jnp.float32),
            pltpu.VMEM((TOPK, GRP), jnp.float32),
            pltpu.VMEM((TOPK,), jnp.float32),
            pltpu.VMEM((TOPK,), jnp.float32),
            pltpu.VMEM((TOPK,), jnp.int32),
            pltpu.VMEM((TOPK,), jnp.int32),
            pltpu.VMEM((TOPK, D_IN), jnp.float32),
            pltpu.VMEM((TOPK, D_IN), jnp.float32),
            pltpu.VMEM((D_IN,), jnp.float32),
            pltpu.VMEM((D_IN,), jnp.float32),
            pltpu.SemaphoreType.DMA,
            pltpu.SemaphoreType.DMA,
            pltpu.SemaphoreType.DMA,
            pltpu.SemaphoreType.DMA,
            pltpu.SemaphoreType.DMA,
            pltpu.SemaphoreType.DMA,
        ],
    )(_sc_body)
    return f(pa_rows, vals, gids, W_dec, b_dec)


def kernel(x, W_enc, b_enc, W_dec, b_dec):
    sae_in = x - b_dec[None, :]
    pre_acts, gm128_3d = _encode(sae_in, W_enc, b_enc)
    gm128 = gm128_3d.transpose(1, 0, 2).reshape(N_TOK, NGRP)
    vals, gids = _group_select(gm128)
    top_acts, top_idx, sae_out = _sc_select_decode(
        pre_acts, vals, gids, W_dec, b_dec)
    e = x - sae_out
    total_variance = jnp.sum((x - jnp.mean(x, axis=0)) ** 2)
    fvu = jnp.sum(e * e) / total_variance
    return sae_out, top_acts, top_idx, fvu


# speedup vs baseline: 9.3660x; 1.0050x over previous
import functools

import jax
import jax.numpy as jnp
from jax import lax
from jax.experimental import pallas as pl
from jax.experimental.pallas import tpu as pltpu
from jax.experimental.pallas import tpu_sc as plsc

D_IN = 768
N_LAT = 49152
N_TOK = 2048
TOPK = 64
GRP = 128
NGRP = N_LAT // GRP
CHK = 16
NCHK = GRP // CHK


_LAT_BLK = 512
_GPB = _LAT_BLK // GRP
_CPB = _LAT_BLK // CHK


def _enc_body(x_ref, w_ref, b_ref, out_ref, gm128_ref):
    acts = lax.dot_general(
        x_ref[...], w_ref[...],
        dimension_numbers=(((1,), (1,)), ((), ())),
        preferred_element_type=jnp.float32,
    )
    acts = jnp.maximum(acts + b_ref[...], 0.0)
    out_ref[...] = acts
    gm128_ref[...] = jnp.max(
        acts.reshape(N_TOK, _GPB, GRP), axis=2).reshape(1, N_TOK, _GPB)


def _encode(sae_in, W_enc, b_enc):
    grid = (N_LAT // _LAT_BLK,)
    return pl.pallas_call(
        _enc_body,
        grid=grid,
        in_specs=[
            pl.BlockSpec((N_TOK, D_IN), lambda j: (0, 0)),
            pl.BlockSpec((_LAT_BLK, D_IN), lambda j: (j, 0)),
            pl.BlockSpec((1, _LAT_BLK), lambda j: (0, j)),
        ],
        out_specs=[
            pl.BlockSpec((N_TOK, _LAT_BLK), lambda j: (0, j)),
            pl.BlockSpec((1, N_TOK, _GPB), lambda j: (j, 0, 0)),
        ],
        out_shape=[
            jax.ShapeDtypeStruct((N_TOK, N_LAT), jnp.float32),
            jax.ShapeDtypeStruct((N_LAT // _LAT_BLK, N_TOK, _GPB),
                                 jnp.float32),
        ],
    )(sae_in, W_enc, b_enc.reshape(1, N_LAT))


_TB = 256


def _gsel_body(gm_ref, vals_ref, gids_ref):
    gm = gm_ref[...]
    lane = lax.broadcasted_iota(jnp.int32, (_TB, NGRP), 1)
    vals = []
    gids = []
    for _ in range(TOPK):
        m = jnp.max(gm, axis=1, keepdims=True)
        hit = gm == m
        idx = jnp.min(jnp.where(hit, lane, NGRP), axis=1, keepdims=True)
        vals.append(m)
        gids.append(idx)
        gm = jnp.where(lane == idx, -1.0, gm)
    vals_ref[...] = jnp.concatenate(vals, axis=1)
    gids_ref[...] = jnp.concatenate(gids, axis=1)


def _group_select(gm128):
    grid = (N_TOK // _TB,)
    return pl.pallas_call(
        _gsel_body,
        grid=grid,
        in_specs=[pl.BlockSpec((_TB, NGRP), lambda i: (i, 0))],
        out_specs=[
            pl.BlockSpec((_TB, TOPK), lambda i: (i, 0)),
            pl.BlockSpec((_TB, TOPK), lambda i: (i, 0)),
        ],
        out_shape=[
            jax.ShapeDtypeStruct((N_TOK, TOPK), jnp.float32),
            jax.ShapeDtypeStruct((N_TOK, TOPK), jnp.int32),
        ],
    )(gm128)


_NC = 2
_NS = 16
_NW = _NC * _NS
_TPW = N_TOK // _NW
_CCH = D_IN // 16


def _sc_body(pa_hbm, vals_hbm, gids_hbm, wdec_hbm, bdec_hbm,
             topv_hbm, topi_hbm, out_hbm,
             gidslab_v, gid_v, cur0_v, cur1_v, rowidx_v,
             vals0_v, vals1_v, topv0_v, topv1_v, topi0_v, topi1_v,
             wrows0_v, wrows1_v, bdec_v, outrow_v,
             pa_sem0, pa_sem1, wd_sem0, wd_sem1, cu_sem0, cu_sem1,
             em_sem0, em_sem1, ow_sem):
    wid = lax.axis_index("s") * _NC + lax.axis_index("c")
    base = wid * _TPW
    pltpu.sync_copy(bdec_hbm, bdec_v)
    pltpu.sync_copy(gids_hbm.at[pl.ds(base, _TPW)], gidslab_v)
    lanes16 = lax.iota(jnp.int32, 16)
    big = jnp.int32(255)
    vals_b = (vals0_v, vals1_v)
    cur_b = (cur0_v, cur1_v)
    topv_b = (topv0_v, topv1_v)
    topi_b = (topi0_v, topi1_v)
    wrows_b = (wrows0_v, wrows1_v)
    pa_sems = (pa_sem0, pa_sem1)
    wd_sems = (wd_sem0, wd_sem1)
    cu_sems = (cu_sem0, cu_sem1)
    em_sems = (em_sem0, em_sem1)

    def _extract(ref, pos):
        seg = ref[pl.ds((pos >> 4) * 16, 16)]
        return jnp.max(jnp.where(lanes16 == (pos & 15), seg,
                                 jnp.full((16,), -2147483647,
                                          seg.dtype)))

    def _insert(ref, pos, val):
        off = (pos >> 4) * 16
        seg = ref[pl.ds(off, 16)]
        ref[pl.ds(off, 16)] = jnp.where(lanes16 == (pos & 15),
                                        jnp.full((16,), val, seg.dtype), seg)

    def issue_pa(tl, buf):
        tok = base + tl
        for j in range(4):
            rowidx_v[pl.ds(j * 16, 16)] = (
                gidslab_v[tl, pl.ds(j * 16, 16)] + tok * NGRP)
        pltpu.async_copy(vals_hbm.at[tok], cur_b[buf], cu_sems[buf])
        return pltpu.async_copy(pa_hbm.at[rowidx_v], vals_b[buf],
                                pa_sems[buf])

    def pops(tl, buf):
        vals_v = vals_b[buf]
        cur_v = cur_b[buf]
        topv_v = topv_b[buf]
        topi_v = topi_b[buf]

        @pl.when(tl >= 2)
        def _():
            pltpu.make_async_copy(topv_v, topv_hbm.at[base],
                                  em_sems[buf]).wait()
            pltpu.make_async_copy(topi_v, topi_hbm.at[base],
                                  em_sems[buf]).wait()

        for j in range(4):
            gid_v[pl.ds(j * 16, 16)] = gidslab_v[tl, pl.ds(j * 16, 16)]

        bigi = jnp.int32(1 << 30)

        def pop_body(i, carry2):
            c0 = cur_v[pl.ds(0, 16)]
            c1 = cur_v[pl.ds(16, 16)]
            c2 = cur_v[pl.ds(32, 16)]
            c3 = cur_v[pl.ds(48, 16)]
            v = jnp.max(jnp.maximum(jnp.maximum(c0, c1),
                                    jnp.maximum(c2, c3)))
            sc = jnp.full((16,), bigi, jnp.int32)
            for j, cj in enumerate((c0, c1, c2, c3)):
                sc = jnp.minimum(sc, jnp.where(cj == v,
                                               j * 16 + lanes16, bigi))
            slot = jnp.minimum(jnp.min(sc), jnp.int32(TOPK - 1))
            lc = jnp.full((16,), bigi, jnp.int32)
            for c in range(NCHK):
                ch = vals_v[slot, pl.ds(c * CHK, CHK)]
                lc = jnp.minimum(lc, jnp.where(ch == v,
                                               c * 256 + lanes16, bigi))
            code = jnp.min(lc)
            cbest = jnp.minimum(code >> 8, jnp.int32(NCHK - 1))
            lane = code & 15
            gsel = _extract(gid_v, slot)
            gidx = gsel * GRP + cbest * CHK + lane
            _insert(topv_v, i, v)
            _insert(topi_v, i, gidx)
            ch = vals_v[slot, pl.ds(cbest * CHK, CHK)]
            vals_v[slot, pl.ds(cbest * CHK, CHK)] = jnp.where(
                lanes16 == lane, jnp.float32(-1.0), ch)
            ncur = jnp.full((16,), -1.0, jnp.float32)
            for c in range(NCHK):
                ncur = jnp.maximum(ncur, vals_v[slot, pl.ds(c * CHK, CHK)])
            _insert(cur_v, slot, jnp.max(ncur))
            return carry2

        lax.fori_loop(0, TOPK, pop_body, 0, unroll=False)
        tok = base + tl
        pltpu.async_copy(topv_v, topv_hbm.at[tok], em_sems[buf])
        pltpu.async_copy(topi_v, topi_hbm.at[tok], em_sems[buf])

    def accumulate(tl, buf):
        topv_v = topv_b[buf]
        wrows_v = wrows_b[buf]

        @pl.when(tl >= 1)
        def _():
            pltpu.make_async_copy(outrow_v, out_hbm.at[base], ow_sem).wait()

        for c in range(_CCH):
            outrow_v[pl.ds(c * 16, 16)] = bdec_v[pl.ds(c * 16, 16)]

        def k_body(k, carry2):
            a_b = jnp.full((16,), _extract(topv_v, k), jnp.float32)
            for c in range(_CCH):
                plsc.addupdate(outrow_v.at[pl.ds(c * 16, 16)],
                               a_b * wrows_v[k, pl.ds(c * 16, 16)])
            return carry2

        lax.fori_loop(0, TOPK, k_body, 0, unroll=False)
        pltpu.async_copy(outrow_v, out_hbm.at[base + tl], ow_sem)

    def wait_pa(buf):
        pltpu.make_async_copy(vals_hbm.at[base], cur_b[buf],
                              cu_sems[buf]).wait()
        pltpu.make_async_copy(pa_hbm.at[rowidx_v], vals_b[buf],
                              pa_sems[buf]).wait()

    def wait_wd(buf):
        pltpu.make_async_copy(wdec_hbm.at[topi_b[buf]], wrows_b[buf],
                              wd_sems[buf]).wait()

    def issue_wd(buf):
        pltpu.async_copy(wdec_hbm.at[topi_b[buf]], wrows_b[buf],
                         wd_sems[buf])

    issue_pa(0, 0)
    wait_pa(0)
    pops(jnp.int32(0), 0)
    issue_wd(0)
    issue_pa(1, 1)

    def step(t, buf):
        wait_pa(buf)
        pops(t, buf)
        issue_wd(buf)
        issue_pa(jnp.minimum(t + 1, _TPW - 1), 1 - buf)
        wait_wd(1 - buf)
        accumulate(t - 1, 1 - buf)

    def loop_body(i, carry):
        step(1 + 2 * i, 1)
        step(2 + 2 * i, 0)
        return carry

    lax.fori_loop(0, (_TPW - 2) // 2, loop_body, 0, unroll=False)
    wait_pa(1)
    pops(jnp.int32(_TPW - 1), 1)
    issue_wd(1)
    wait_wd(0)
    accumulate(jnp.int32(_TPW - 2), 0)
    wait_wd(1)
    accumulate(jnp.int32(_TPW - 1), 1)
    for buf in (0, 1):
        pltpu.make_async_copy(topv_b[buf], topv_hbm.at[base],
                              em_sems[buf]).wait()
        pltpu.make_async_copy(topi_b[buf], topi_hbm.at[base],
                              em_sems[buf]).wait()
    pltpu.make_async_copy(outrow_v, out_hbm.at[base], ow_sem).wait()


def _sc_select_decode(pre_acts, vals, gids, W_dec, b_dec):
    pa_rows = pre_acts.reshape(N_TOK * NGRP, GRP)
    mesh = plsc.VectorSubcoreMesh(core_axis_name="c", subcore_axis_name="s")
    f = functools.partial(
        pl.kernel,
        mesh=mesh,
        compiler_params=pltpu.CompilerParams(needs_layout_passes=False),
        out_type=[
            jax.ShapeDtypeStruct((N_TOK, TOPK), jnp.float32),
            jax.ShapeDtypeStruct((N_TOK, TOPK), jnp.int32),
            jax.ShapeDtypeStruct((N_TOK, D_IN), jnp.float32),
        ],
        scratch_types=[
            pltpu.VMEM((_TPW, TOPK), jnp.int32),
            pltpu.VMEM((TOPK,), jnp.int32),
            pltpu.VMEM((TOPK,), jnp.float32),
            pltpu.VMEM((TOPK,), jnp.float32),
            pltpu.VMEM((TOPK,), jnp.int32),
            pltpu.VMEM((TOPK, GRP), jnp.float32),
            pltpu.VMEM((TOPK, GRP), jnp.float32),
            pltpu.VMEM((TOPK,), jnp.float32),
            pltpu.VMEM((TOPK,), jnp.float32),
            pltpu.VMEM((TOPK,), jnp.int32),
            pltpu.VMEM((TOPK,), jnp.int32),
            pltpu.VMEM((TOPK, D_IN), jnp.float32),
            pltpu.VMEM((TOPK, D_IN), jnp.float32),
            pltpu.VMEM((D_IN,), jnp.float32),
            pltpu.VMEM((D_IN,), jnp.float32),
            pltpu.SemaphoreType.DMA,
            pltpu.SemaphoreType.DMA,
            pltpu.SemaphoreType.DMA,
            pltpu.SemaphoreType.DMA,
            pltpu.SemaphoreType.DMA,
            pltpu.SemaphoreType.DMA,
            pltpu.SemaphoreType.DMA,
            pltpu.SemaphoreType.DMA,
            pltpu.SemaphoreType.DMA,
        ],
    )(_sc_body)
    return f(pa_rows, vals, gids, W_dec, b_dec)


def kernel(x, W_enc, b_enc, W_dec, b_dec):
    sae_in = x - b_dec[None, :]
    pre_acts, gm128_3d = _encode(sae_in, W_enc, b_enc)
    gm128 = gm128_3d.transpose(1, 0, 2).reshape(N_TOK, NGRP)
    vals, gids = _group_select(gm128)
    top_acts, top_idx, sae_out = _sc_select_decode(
        pre_acts, vals, gids, W_dec, b_dec)
    e = x - sae_out
    total_variance = jnp.sum((x - jnp.mean(x, axis=0)) ** 2)
    fvu = jnp.sum(e * e) / total_variance
    return sae_out, top_acts, top_idx, fvu
